# baseline (device time: 53327 ns/iter reference)
import jax
import jax.numpy as jnp
from jax import lax
from jax.experimental import pallas as pl
from jax.experimental.pallas import tpu as pltpu

B = 16
NB = 128
BS = 16
H = 16
D = 64
P_LOCAL = 128
T_LOCAL = P_LOCAL * BS
G = H // 2
M2 = 2 * B
D2 = 2 * D
SCALE = D ** -0.5
NEG = -1e30


def kernel(Q, K, V, bt, lens):
    q3 = (Q.reshape(B, H, D) * SCALE).transpose(1, 0, 2)
    q4 = q3.reshape(G, 2, B, D)
    eye2 = jnp.eye(2, dtype=q4.dtype)
    qblk = (q4[:, :, :, None, :] * eye2[None, :, None, :, None])
    qblk = qblk.reshape(G, M2, D2).astype(jnp.bfloat16)
    k2 = K.reshape(T_LOCAL, H * D).astype(jnp.bfloat16)
    v2 = V.reshape(T_LOCAL, H * D).astype(jnp.bfloat16)
    lens2 = lens.reshape(B, 1)

    def body(q_ref, k_ref, v_ref, bt_ref, lens_ref, out_ref,
             o_send, st_send, o_recv, st_recv, send_sems, recv_sems):
        my_x = lax.axis_index("x")
        my_y = lax.axis_index("y")
        peer = (1 - my_x, my_y)

        barrier = pltpu.get_barrier_semaphore()
        pl.semaphore_signal(barrier, inc=1, device_id=peer,
                            device_id_type=pl.DeviceIdType.MESH)
        pl.semaphore_wait(barrier, 1)

        x_off = my_x * P_LOCAL
        bt_arr = bt_ref[...]
        lens_arr = lens_ref[...]
        slot = lax.broadcasted_iota(jnp.int32, (B, NB, P_LOCAL), 1)
        page = lax.broadcasted_iota(jnp.int32, (B, NB, P_LOCAL), 2)
        hit = (bt_arr[:, :, None] == page + x_off) & (
            slot < lens_arr[:, :, None])
        w = jnp.sum(hit.astype(jnp.float32), axis=1)
        logw = jnp.where(w > 0, jnp.log(w), NEG).astype(jnp.bfloat16)

        tpage = lax.broadcasted_iota(jnp.int32, (P_LOCAL, T_LOCAL), 1) // BS
        prow = lax.broadcasted_iota(jnp.int32, (P_LOCAL, T_LOCAL), 0)
        expand = (tpage == prow).astype(jnp.bfloat16)
        logw_tok = lax.dot_general(
            logw, expand, (((1,), (0,)), ((), ())),
            preferred_element_type=jnp.float32,
        )
        logw32 = jnp.concatenate([logw_tok, logw_tok], axis=0)

        for g in range(G):
            lanes = slice(g * D2, (g + 1) * D2)
            s = lax.dot_general(
                q_ref[g], k_ref[:, lanes], (((1,), (1,)), ((), ())),
                preferred_element_type=jnp.float32,
            ) + logw32
            m_g = jnp.max(s, axis=1, keepdims=True)
            p_un = jnp.exp((s - m_g).astype(jnp.bfloat16))
            l_g = jnp.sum(p_un, axis=1, keepdims=True,
                          dtype=jnp.float32)
            o_g = lax.dot_general(
                p_un, v_ref[:, lanes], (((1,), (0,)), ((), ())),
                preferred_element_type=jnp.float32,
            )
            o_send[2 * g] = o_g[0:B, 0:D]
            o_send[2 * g + 1] = o_g[B:M2, D:D2]
            st_send[2 * g] = jnp.concatenate(
                [m_g[0:B], l_g[0:B]], axis=1)
            st_send[2 * g + 1] = jnp.concatenate(
                [m_g[B:M2], l_g[B:M2]], axis=1)

        rdma_o = pltpu.make_async_remote_copy(
            src_ref=o_send, dst_ref=o_recv,
            send_sem=send_sems.at[0], recv_sem=recv_sems.at[0],
            device_id=peer, device_id_type=pl.DeviceIdType.MESH,
        )
        rdma_st = pltpu.make_async_remote_copy(
            src_ref=st_send, dst_ref=st_recv,
            send_sem=send_sems.at[1], recv_sem=recv_sems.at[1],
            device_id=peer, device_id_type=pl.DeviceIdType.MESH,
        )
        rdma_o.start()
        rdma_st.start()
        rdma_o.wait()
        rdma_st.wait()

        m_loc = st_send[:, :, 0:1]
        l_loc = st_send[:, :, 1:2]
        m_p = st_recv[:, :, 0:1]
        l_p = st_recv[:, :, 1:2]
        m_new = jnp.maximum(m_loc, m_p)
        a = jnp.exp(m_loc - m_new)
        c = jnp.exp(m_p - m_new)
        l_new = l_loc * a + l_p * c
        out_ref[...] = (o_send[...] * a + o_recv[...] * c) / l_new

    out = pl.pallas_call(
        body,
        out_shape=jax.ShapeDtypeStruct((H, B, D), jnp.float32),
        in_specs=[
            pl.BlockSpec(memory_space=pltpu.VMEM),
            pl.BlockSpec(memory_space=pltpu.VMEM),
            pl.BlockSpec(memory_space=pltpu.VMEM),
            pl.BlockSpec(memory_space=pltpu.VMEM),
            pl.BlockSpec(memory_space=pltpu.VMEM),
        ],
        out_specs=pl.BlockSpec(memory_space=pltpu.VMEM),
        scratch_shapes=[
            pltpu.VMEM((H, B, D), jnp.float32),
            pltpu.VMEM((H, B, 2), jnp.float32),
            pltpu.VMEM((H, B, D), jnp.float32),
            pltpu.VMEM((H, B, 2), jnp.float32),
            pltpu.SemaphoreType.DMA((2,)),
            pltpu.SemaphoreType.DMA((2,)),
        ],
        compiler_params=pltpu.CompilerParams(collective_id=0),
    )(qblk, k2, v2, bt, lens2)

    return out.swapaxes(0, 1).reshape(B, 1, H, D)


# device time: 42540 ns/iter; 1.2536x vs baseline; 1.2536x over previous
import jax
import jax.numpy as jnp
from jax import lax
from jax.experimental import pallas as pl
from jax.experimental.pallas import tpu as pltpu

B = 16
NB = 128
BS = 16
H = 16
D = 64
P_LOCAL = 128
T_LOCAL = P_LOCAL * BS
G = H // 2
M2 = 2 * B
D2 = 2 * D
SCALE = D ** -0.5
NEG = -1e30


def kernel(Q, K, V, bt, lens):
    q = (Q.reshape(B, H, D) * SCALE).astype(jnp.bfloat16).swapaxes(0, 1)
    k2 = (K.reshape(T_LOCAL, G, 2, D).astype(jnp.bfloat16)
          .transpose(1, 0, 2, 3).reshape(G, T_LOCAL, D2))
    v2 = (V.reshape(T_LOCAL, G, 2, D).astype(jnp.bfloat16)
          .transpose(1, 0, 2, 3).reshape(G, T_LOCAL, D2))
    lens2 = lens.reshape(B, 1)

    def body(q_ref, k_ref, v_ref, bt_ref, lens_ref, out_ref,
             o_send, st_send, o_recv, st_recv, send_sems, recv_sems):
        my_x = lax.axis_index("x")
        my_y = lax.axis_index("y")
        peer = (1 - my_x, my_y)

        barrier = pltpu.get_barrier_semaphore()
        pl.semaphore_signal(barrier, inc=1, device_id=peer,
                            device_id_type=pl.DeviceIdType.MESH)
        pl.semaphore_wait(barrier, 1)

        x_off = my_x * P_LOCAL
        bt_arr = bt_ref[...]
        lens_arr = lens_ref[...]
        slot = lax.broadcasted_iota(jnp.int32, (B, NB, P_LOCAL), 1)
        page = lax.broadcasted_iota(jnp.int32, (B, NB, P_LOCAL), 2)
        hit = (bt_arr[:, :, None] == page + x_off) & (
            slot < lens_arr[:, :, None])
        w = jnp.sum(hit.astype(jnp.float32), axis=1)
        logw = jnp.where(w > 0, jnp.log(w), NEG).astype(jnp.bfloat16)

        tpage = lax.broadcasted_iota(jnp.int32, (P_LOCAL, T_LOCAL), 1) // BS
        prow = lax.broadcasted_iota(jnp.int32, (P_LOCAL, T_LOCAL), 0)
        expand = (tpage == prow).astype(jnp.bfloat16)
        logw_tok = lax.dot_general(
            logw, expand, (((1,), (0,)), ((), ())),
            preferred_element_type=jnp.float32,
        )
        logw32 = jnp.concatenate([logw_tok, logw_tok], axis=0)

        zeros_bd = jnp.zeros((B, D), dtype=jnp.bfloat16)
        for g in range(G):
            qb = jnp.concatenate([
                jnp.concatenate([q_ref[2 * g], zeros_bd], axis=1),
                jnp.concatenate([zeros_bd, q_ref[2 * g + 1]], axis=1),
            ], axis=0)
            s = lax.dot_general(
                qb, k_ref[g], (((1,), (1,)), ((), ())),
                preferred_element_type=jnp.float32,
            ) + logw32
            m_g = jnp.max(s, axis=1, keepdims=True)
            p_un = jnp.exp((s - m_g).astype(jnp.bfloat16))
            l_g = jnp.sum(p_un, axis=1, keepdims=True,
                          dtype=jnp.float32)
            o_g = lax.dot_general(
                p_un, v_ref[g], (((1,), (0,)), ((), ())),
                preferred_element_type=jnp.float32,
            )
            o_send[2 * g] = o_g[0:B, 0:D]
            o_send[2 * g + 1] = o_g[B:M2, D:D2]
            st_send[2 * g] = jnp.concatenate(
                [m_g[0:B], l_g[0:B]], axis=1)
            st_send[2 * g + 1] = jnp.concatenate(
                [m_g[B:M2], l_g[B:M2]], axis=1)

        rdma_o = pltpu.make_async_remote_copy(
            src_ref=o_send, dst_ref=o_recv,
            send_sem=send_sems.at[0], recv_sem=recv_sems.at[0],
            device_id=peer, device_id_type=pl.DeviceIdType.MESH,
        )
        rdma_st = pltpu.make_async_remote_copy(
            src_ref=st_send, dst_ref=st_recv,
            send_sem=send_sems.at[1], recv_sem=recv_sems.at[1],
            device_id=peer, device_id_type=pl.DeviceIdType.MESH,
        )
        rdma_o.start()
        rdma_st.start()
        rdma_o.wait()
        rdma_st.wait()

        m_loc = st_send[:, :, 0:1]
        l_loc = st_send[:, :, 1:2]
        m_p = st_recv[:, :, 0:1]
        l_p = st_recv[:, :, 1:2]
        m_new = jnp.maximum(m_loc, m_p)
        a = jnp.exp(m_loc - m_new)
        c = jnp.exp(m_p - m_new)
        l_new = l_loc * a + l_p * c
        out_ref[...] = (o_send[...] * a + o_recv[...] * c) / l_new

    out = pl.pallas_call(
        body,
        out_shape=jax.ShapeDtypeStruct((H, B, D), jnp.float32),
        in_specs=[
            pl.BlockSpec(memory_space=pltpu.VMEM),
            pl.BlockSpec(memory_space=pltpu.VMEM),
            pl.BlockSpec(memory_space=pltpu.VMEM),
            pl.BlockSpec(memory_space=pltpu.VMEM),
            pl.BlockSpec(memory_space=pltpu.VMEM),
        ],
        out_specs=pl.BlockSpec(memory_space=pltpu.VMEM),
        scratch_shapes=[
            pltpu.VMEM((H, B, D), jnp.float32),
            pltpu.VMEM((H, B, 2), jnp.float32),
            pltpu.VMEM((H, B, D), jnp.float32),
            pltpu.VMEM((H, B, 2), jnp.float32),
            pltpu.SemaphoreType.DMA((2,)),
            pltpu.SemaphoreType.DMA((2,)),
        ],
        compiler_params=pltpu.CompilerParams(collective_id=0),
    )(q, k2, v2, bt, lens2)

    return out.swapaxes(0, 1).reshape(B, 1, H, D)


# device time: 31010 ns/iter; 1.7197x vs baseline; 1.3718x over previous
import jax
import jax.numpy as jnp
from jax import lax
from jax.experimental import pallas as pl
from jax.experimental.pallas import tpu as pltpu

B = 16
NB = 128
BS = 16
H = 16
D = 64
P_LOCAL = 128
T_LOCAL = P_LOCAL * BS
G = H // 2
M2 = 2 * B
D2 = 2 * D
SCALE = D ** -0.5
NEG = -1e30


def kernel(Q, K, V, bt, lens):
    q = (Q.reshape(B, H, D) * SCALE).astype(jnp.bfloat16).swapaxes(0, 1)
    k2 = K.reshape(T_LOCAL, H, D).astype(jnp.bfloat16).swapaxes(0, 1)
    v2 = V.reshape(T_LOCAL, H, D).astype(jnp.bfloat16).swapaxes(0, 1)
    lens2 = lens.reshape(B, 1)

    def body(q_ref, k_ref, v_ref, bt_ref, lens_ref, out_ref,
             o_send, st_send, o_recv, st_recv, send_sems, recv_sems):
        my_x = lax.axis_index("x")
        my_y = lax.axis_index("y")
        peer = (1 - my_x, my_y)

        barrier = pltpu.get_barrier_semaphore()
        pl.semaphore_signal(barrier, inc=1, device_id=peer,
                            device_id_type=pl.DeviceIdType.MESH)
        pl.semaphore_wait(barrier, 1)

        x_off = my_x * P_LOCAL
        bt_arr = bt_ref[...]
        lens_arr = lens_ref[...]
        slot = lax.broadcasted_iota(jnp.int32, (B, NB, P_LOCAL), 1)
        page = lax.broadcasted_iota(jnp.int32, (B, NB, P_LOCAL), 2)
        hit = (bt_arr[:, :, None] == page + x_off) & (
            slot < lens_arr[:, :, None])
        w = jnp.sum(hit.astype(jnp.float32), axis=1)
        logw = jnp.where(w > 0, jnp.log(w), NEG).astype(jnp.bfloat16)

        tpage = lax.broadcasted_iota(jnp.int32, (P_LOCAL, T_LOCAL), 1) // BS
        prow = lax.broadcasted_iota(jnp.int32, (P_LOCAL, T_LOCAL), 0)
        expand = (tpage == prow).astype(jnp.bfloat16)
        logw_tok = lax.dot_general(
            logw, expand, (((1,), (0,)), ((), ())),
            preferred_element_type=jnp.float32,
        )
        logw32 = jnp.concatenate([logw_tok, logw_tok], axis=0)

        zeros_bd = jnp.zeros((B, D), dtype=jnp.bfloat16)
        for g in range(G):
            qb = jnp.concatenate([
                jnp.concatenate([q_ref[2 * g], zeros_bd], axis=1),
                jnp.concatenate([zeros_bd, q_ref[2 * g + 1]], axis=1),
            ], axis=0)
            kslab = jnp.concatenate(
                [k_ref[2 * g], k_ref[2 * g + 1]], axis=1)
            s = lax.dot_general(
                qb, kslab, (((1,), (1,)), ((), ())),
                preferred_element_type=jnp.float32,
            ) + logw32
            m_g = jnp.max(s, axis=1, keepdims=True)
            p_un = jnp.exp((s - m_g).astype(jnp.bfloat16))
            l_g = jnp.sum(p_un, axis=1, keepdims=True,
                          dtype=jnp.float32)
            vslab = jnp.concatenate(
                [v_ref[2 * g], v_ref[2 * g + 1]], axis=1)
            o_g = lax.dot_general(
                p_un, vslab, (((1,), (0,)), ((), ())),
                preferred_element_type=jnp.float32,
            )
            o_send[2 * g] = o_g[0:B, 0:D]
            o_send[2 * g + 1] = o_g[B:M2, D:D2]
            st_send[2 * g] = jnp.concatenate(
                [m_g[0:B], l_g[0:B]], axis=1)
            st_send[2 * g + 1] = jnp.concatenate(
                [m_g[B:M2], l_g[B:M2]], axis=1)

        rdma_o = pltpu.make_async_remote_copy(
            src_ref=o_send, dst_ref=o_recv,
            send_sem=send_sems.at[0], recv_sem=recv_sems.at[0],
            device_id=peer, device_id_type=pl.DeviceIdType.MESH,
        )
        rdma_st = pltpu.make_async_remote_copy(
            src_ref=st_send, dst_ref=st_recv,
            send_sem=send_sems.at[1], recv_sem=recv_sems.at[1],
            device_id=peer, device_id_type=pl.DeviceIdType.MESH,
        )
        rdma_o.start()
        rdma_st.start()
        rdma_o.wait()
        rdma_st.wait()

        m_loc = st_send[:, :, 0:1]
        l_loc = st_send[:, :, 1:2]
        m_p = st_recv[:, :, 0:1]
        l_p = st_recv[:, :, 1:2]
        m_new = jnp.maximum(m_loc, m_p)
        a = jnp.exp(m_loc - m_new)
        c = jnp.exp(m_p - m_new)
        l_new = l_loc * a + l_p * c
        out_ref[...] = (o_send[...] * a + o_recv[...] * c) / l_new

    out = pl.pallas_call(
        body,
        out_shape=jax.ShapeDtypeStruct((H, B, D), jnp.float32),
        in_specs=[
            pl.BlockSpec(memory_space=pltpu.VMEM),
            pl.BlockSpec(memory_space=pltpu.VMEM),
            pl.BlockSpec(memory_space=pltpu.VMEM),
            pl.BlockSpec(memory_space=pltpu.VMEM),
            pl.BlockSpec(memory_space=pltpu.VMEM),
        ],
        out_specs=pl.BlockSpec(memory_space=pltpu.VMEM),
        scratch_shapes=[
            pltpu.VMEM((H, B, D), jnp.float32),
            pltpu.VMEM((H, B, 2), jnp.float32),
            pltpu.VMEM((H, B, D), jnp.float32),
            pltpu.VMEM((H, B, 2), jnp.float32),
            pltpu.SemaphoreType.DMA((2,)),
            pltpu.SemaphoreType.DMA((2,)),
        ],
        compiler_params=pltpu.CompilerParams(collective_id=0),
    )(q, k2, v2, bt, lens2)

    return out.swapaxes(0, 1).reshape(B, 1, H, D)
